# baseline (speedup 1.0000x reference)
import functools

import jax
import jax.numpy as jnp
from jax import lax
from jax.experimental import pallas as pl
from jax.experimental.pallas import tpu as pltpu
from jax.experimental.pallas import tpu_sc as plsc

_L = 16


@functools.lru_cache(maxsize=None)
def _build(B, V, D, half):
    NC, NS = 2, 16
    NW = NC * NS
    b_per_w = B // NW
    CH = 128
    n_ch = b_per_w // CH
    NBUF = 2
    UNROLL = 4

    mesh = plsc.VectorSubcoreMesh(core_axis_name="c", subcore_axis_name="s")

    @functools.partial(
        pl.kernel,
        mesh=mesh,
        out_type=(
            jax.ShapeDtypeStruct((B * half,), jnp.float32),
            jax.ShapeDtypeStruct((B * half,), jnp.float32),
        ),
        scratch_types=[
            pltpu.VMEM((b_per_w,), jnp.int32),
            pltpu.VMEM((NBUF, CH, D), jnp.float32),
            pltpu.VMEM((b_per_w * half,), jnp.float32),
            pltpu.VMEM((b_per_w * half,), jnp.float32),
        ]
        + [pltpu.SemaphoreType.DMA] * (NBUF + 2),
    )
    def k(mem, idx_hbm, emb_hbm, num_hbm, idx_v, rows_v, emb_v, num_v, *sems):
        g = sems[0:NBUF]
        sem_on = sems[NBUF]
        sem_oe = sems[NBUF + 1]

        wid = lax.axis_index("s") * NC + lax.axis_index("c")
        base = wid * b_per_w
        pltpu.sync_copy(idx_hbm.at[pl.ds(base, b_per_w)], idx_v)

        handles = {}

        def issue_gather(c):
            buf = c % NBUF
            handles[("g", c)] = pltpu.async_copy(
                mem.at[idx_v.at[pl.ds(c * CH, CH)]], rows_v.at[buf], g[buf])

        issue_gather(0)
        for c in range(n_ch):
            buf = c % NBUF
            cb = c * CH
            handles[("g", c)].wait()
            if c + 1 < n_ch:
                issue_gather(c + 1)

            def body(i, _):
                for r in range(UNROLL):
                    row = UNROLL * i + r
                    rowbase = (cb + row) * half
                    for j in range(half // _L):
                        num = rows_v[buf, row, pl.ds(j * _L, _L)]
                        den = rows_v[buf, row, pl.ds(half + j * _L, _L)]
                        num_v[pl.ds(rowbase + j * _L, _L)] = num
                        emb_v[pl.ds(rowbase + j * _L, _L)] = num / den
                return 0

            lax.fori_loop(0, CH // UNROLL, body, 0)
            handles[("on", c)] = pltpu.async_copy(
                num_v.at[pl.ds(cb * half, CH * half)],
                num_hbm.at[pl.ds((base + cb) * half, CH * half)], sem_on)
            handles[("oe", c)] = pltpu.async_copy(
                emb_v.at[pl.ds(cb * half, CH * half)],
                emb_hbm.at[pl.ds((base + cb) * half, CH * half)], sem_oe)

        for c in range(n_ch):
            handles[("on", c)].wait()
            handles[("oe", c)].wait()

    return k


def kernel(memory, nodes, memory_dim):
    V, D = memory.shape
    B = nodes.shape[0]
    half = D // 2
    k = _build(B, V, D, half)
    emb, num = k(memory, nodes.astype(jnp.int32))
    return (emb.reshape(B, half), num.reshape(B, half))


# speedup vs baseline: 1.5551x; 1.5551x over previous
import functools

import jax
import jax.numpy as jnp
from jax import lax
from jax.experimental import pallas as pl
from jax.experimental.pallas import tpu as pltpu
from jax.experimental.pallas import tpu_sc as plsc

_L = 16


@functools.lru_cache(maxsize=None)
def _build(B, V, D, half):
    NC, NS = 2, 16
    NW = NC * NS
    b_per_w = B // NW
    CH = 128
    n_ch = b_per_w // CH
    NBUF = 2

    mesh = plsc.VectorSubcoreMesh(core_axis_name="c", subcore_axis_name="s")

    @functools.partial(
        pl.kernel,
        mesh=mesh,
        out_type=(
            jax.ShapeDtypeStruct((B, half), jnp.float32),
            jax.ShapeDtypeStruct((B, half), jnp.float32),
        ),
        scratch_types=[
            pltpu.VMEM((b_per_w,), jnp.int32),
            pltpu.VMEM((NBUF, CH, D), jnp.float32),
            pltpu.VMEM((NBUF, CH, half), jnp.float32),
            pltpu.VMEM((NBUF, CH, half), jnp.float32),
        ]
        + [pltpu.SemaphoreType.DMA] * (3 * NBUF),
    )
    def k(mem, idx_hbm, emb_hbm, num_hbm, idx_v, rows_v, emb_v, num_v, *sems):
        g = sems[0:NBUF]
        on = sems[NBUF:2 * NBUF]
        oe = sems[2 * NBUF:3 * NBUF]

        wid = lax.axis_index("s") * NC + lax.axis_index("c")
        base = wid * b_per_w
        pltpu.sync_copy(idx_hbm.at[pl.ds(base, b_per_w)], idx_v)

        handles = {}

        def issue_gather(c):
            buf = c % NBUF
            handles[("g", c)] = pltpu.async_copy(
                mem.at[idx_v.at[pl.ds(c * CH, CH)]], rows_v.at[buf], g[buf])

        issue_gather(0)
        for c in range(n_ch):
            buf = c % NBUF
            cb = c * CH
            handles[("g", c)].wait()
            if c + 1 < n_ch:
                issue_gather(c + 1)
            if c >= NBUF:
                handles[("on", c - NBUF)].wait()
                handles[("oe", c - NBUF)].wait()

            def body(i, _):
                for j in range(half // _L):
                    s = pl.ds(j * _L, _L)
                    num = rows_v[buf, i, s]
                    den = rows_v[buf, i, pl.ds(half + j * _L, _L)]
                    num_v[buf, i, s] = num
                    emb_v[buf, i, s] = num / den
                return 0

            lax.fori_loop(0, CH, body, 0)
            handles[("on", c)] = pltpu.async_copy(
                num_v.at[buf], num_hbm.at[pl.ds(base + cb, CH)], on[buf])
            handles[("oe", c)] = pltpu.async_copy(
                emb_v.at[buf], emb_hbm.at[pl.ds(base + cb, CH)], oe[buf])

        for c in range(n_ch - NBUF, n_ch):
            handles[("on", c)].wait()
            handles[("oe", c)].wait()

    return k


def kernel(memory, nodes, memory_dim):
    V, D = memory.shape
    B = nodes.shape[0]
    half = D // 2
    k = _build(B, V, D, half)
    emb, num = k(memory, nodes.astype(jnp.int32))
    return (emb, num)
